# baseline (device time: 78170 ns/iter reference)
import jax
import jax.numpy as jnp
from jax import lax
from jax.experimental import pallas as pl
from jax.experimental.pallas import tpu as pltpu

N_DEV = 16
M = 1024
N = 1024
QROWS = 256
PROWS = 64
STRIP = 256
NST = 12


def kernel(A, B):
    def body(a_ref, b_ref, out_ref,
             cAR1, cAL1, cBR1, cBL1, cAR2, cAL2, cBR2, cBL2,
             sAR, rAR, sAL, rAL, sBR, rBR, sBL, rBL):
        my = lax.axis_index("i")
        z4 = lax.div(my, 4)
        q4 = lax.rem(my, 4)

        def m4(v):
            return lax.rem(v + 8, 4)

        pr = z4 * 4 + m4(q4 + 1)
        plq = z4 * 4 + m4(q4 - 1)
        zr = m4(z4 + 1) * 4 + q4
        zl = m4(z4 - 1) * 4 + q4

        cols = {
            "AR": pl.ds(0 * STRIP, STRIP),
            "AL": pl.ds(1 * STRIP, STRIP),
            "BR": pl.ds(2 * STRIP, STRIP),
            "BL": pl.ds(3 * STRIP, STRIP),
        }
        sems = {"AR": (sAR, rAR), "AL": (sAL, rAL),
                "BR": (sBR, rBR), "BL": (sBL, rBL)}
        comm1 = {"AR": cAR1, "AL": cAL1, "BR": cBR1, "BL": cBL1}
        comm2 = {"AR": cAR2, "AL": cAL2, "BR": cBR2, "BL": cBL2}
        geom = {
            "AR": (q4, pr, z4, zr, +1),
            "AL": (q4, plq, z4, zl, -1),
            "BR": (z4, zr, q4, pr, +1),
            "BL": (z4, zl, q4, plq, -1),
        }
        qown = {k: m4(geom[k][0] + geom[k][4]) for k in geom}

        def qrows(qi):
            return pl.ds(qi * QROWS, QROWS)

        def prow(qi, pi):
            return pl.ds(qi * QROWS + pi * PROWS, PROWS)

        def rdma(src, dst, ch, s, tgt):
            snd, rcv = sems[ch]
            return pltpu.make_async_remote_copy(
                src_ref=src, dst_ref=dst,
                send_sem=snd.at[s], recv_sem=rcv.at[s],
                device_id=(tgt,), device_id_type=pl.DeviceIdType.MESH)

        def compute_quarter(idx):
            out_ref[qrows(idx), :] = jnp.dot(
                a_ref[qrows(idx), :], b_ref[:, :],
                preferred_element_type=jnp.float32)

        barrier_sem = pltpu.get_barrier_semaphore()
        for nbr in (pr, plq, zr, zl):
            pl.semaphore_signal(barrier_sem, inc=1, device_id=(nbr,),
                                device_id_type=pl.DeviceIdType.MESH)
        pl.semaphore_wait(barrier_sem, 4)

        compute_quarter(q4)
        pl.when(z4 != q4)(lambda: compute_quarter(z4))

        for s in range(3):
            started = []
            for ch in ("AR", "AL", "BR", "BL"):
                v, tgt, _, _, d = geom[ch]
                sq = m4(v - d * s)
                started.append(
                    rdma(out_ref.at[qrows(sq), cols[ch]], comm1[ch].at[s],
                         ch, s, tgt))
            for r in started:
                r.start()
            if s == 0:
                for j in range(4):
                    pl.when((j != q4) & (j != z4))(
                        lambda j=j: compute_quarter(j))
            for r in started:
                r.wait()
            for ch in ("AR", "AL", "BR", "BL"):
                v, _, _, _, d = geom[ch]
                rq = m4(v - d * (s + 1))
                out_ref[qrows(rq), cols[ch]] = (
                    out_ref[qrows(rq), cols[ch]] + comm1[ch][s])

        for t in range(3):
            s = 3 + t
            started = []
            for ch in ("AR", "AL", "BR", "BL"):
                _, _, v2, tgt2, d = geom[ch]
                sp = m4(v2 - d * t)
                started.append(
                    rdma(out_ref.at[prow(qown[ch], sp), cols[ch]],
                         comm2[ch].at[t], ch, s, tgt2))
            for r in started:
                r.start()
            for r in started:
                r.wait()
            for ch in ("AR", "AL", "BR", "BL"):
                _, _, v2, _, d = geom[ch]
                rp = m4(v2 - d * (t + 1))
                out_ref[prow(qown[ch], rp), cols[ch]] = (
                    out_ref[prow(qown[ch], rp), cols[ch]] + comm2[ch][t])

        for t in range(3):
            s = 6 + t
            started = []
            for ch in ("AR", "AL", "BR", "BL"):
                _, _, v2, tgt2, d = geom[ch]
                gp = m4(v2 + d * (1 - t))
                ref = out_ref.at[prow(qown[ch], gp), cols[ch]]
                started.append(rdma(ref, ref, ch, s, tgt2))
            for r in started:
                r.start()
            for r in started:
                r.wait()

        for t in range(3):
            s = 9 + t
            started = []
            for ch in ("AR", "AL", "BR", "BL"):
                v, tgt, _, _, d = geom[ch]
                gq = m4(v + d * (1 - t))
                ref = out_ref.at[qrows(gq), cols[ch]]
                started.append(rdma(ref, ref, ch, s, tgt))
            for r in started:
                r.start()
            for r in started:
                r.wait()

    return pl.pallas_call(
        body,
        out_shape=jax.ShapeDtypeStruct((M, N), jnp.float32),
        in_specs=[
            pl.BlockSpec(memory_space=pltpu.VMEM),
            pl.BlockSpec(memory_space=pltpu.VMEM),
        ],
        out_specs=pl.BlockSpec(memory_space=pltpu.VMEM),
        scratch_shapes=(
            [pltpu.VMEM((3, QROWS, STRIP), jnp.float32) for _ in range(4)]
            + [pltpu.VMEM((3, PROWS, STRIP), jnp.float32) for _ in range(4)]
            + [pltpu.SemaphoreType.DMA((NST,)) for _ in range(8)]
        ),
        compiler_params=pltpu.CompilerParams(collective_id=0),
    )(A, B)


# device time: 66633 ns/iter; 1.1731x vs baseline; 1.1731x over previous
import jax
import jax.numpy as jnp
from jax import lax
from jax.experimental import pallas as pl
from jax.experimental.pallas import tpu as pltpu

N_DEV = 16
M = 1024
N = 1024
QROWS = 256
PROWS = 64
STRIP = 256
NST = 12
CHS = ("AR", "AL", "BR", "BL")


def kernel(A, B):
    def body(a_ref, b_ref, out_ref,
             cAR1, cAL1, cBR1, cBL1, cAR2, cAL2, cBR2, cBL2,
             sAR, rAR, sAL, rAL, sBR, rBR, sBL, rBL):
        my = lax.axis_index("i")
        z4 = lax.div(my, 4)
        q4 = lax.rem(my, 4)

        def m4(v):
            return lax.rem(v + 8, 4)

        pr = z4 * 4 + m4(q4 + 1)
        plq = z4 * 4 + m4(q4 - 1)
        zr = m4(z4 + 1) * 4 + q4
        zl = m4(z4 - 1) * 4 + q4

        cols = {ch: pl.ds(k * STRIP, STRIP) for k, ch in enumerate(CHS)}
        sems = {"AR": (sAR, rAR), "AL": (sAL, rAL),
                "BR": (sBR, rBR), "BL": (sBL, rBL)}
        comm1 = {"AR": cAR1, "AL": cAL1, "BR": cBR1, "BL": cBL1}
        comm2 = {"AR": cAR2, "AL": cAL2, "BR": cBR2, "BL": cBL2}
        geom = {
            "AR": (q4, pr, z4, zr, +1),
            "AL": (q4, plq, z4, zl, -1),
            "BR": (z4, zr, q4, pr, +1),
            "BL": (z4, zl, q4, plq, -1),
        }
        qown = {k: m4(geom[k][0] + geom[k][4]) for k in geom}

        def qrows(qi):
            return pl.ds(qi * QROWS, QROWS)

        def prow(qi, pi):
            return pl.ds(qi * QROWS + pi * PROWS, PROWS)

        def step_rdma(ch, s):
            v, tgt, v2, tgt2, d = geom[ch]
            snd, rcv = sems[ch]
            if s < 3:
                src = out_ref.at[qrows(m4(v - d * s)), cols[ch]]
                dst = comm1[ch].at[s]
                t = tgt
            elif s < 6:
                src = out_ref.at[prow(qown[ch], m4(v2 - d * (s - 3))), cols[ch]]
                dst = comm2[ch].at[s - 3]
                t = tgt2
            elif s < 9:
                src = out_ref.at[prow(qown[ch], m4(v2 + d * (7 - s))), cols[ch]]
                dst = src
                t = tgt2
            else:
                src = out_ref.at[qrows(m4(v + d * (10 - s))), cols[ch]]
                dst = src
                t = tgt
            return pltpu.make_async_remote_copy(
                src_ref=src, dst_ref=dst,
                send_sem=snd.at[s], recv_sem=rcv.at[s],
                device_id=(t,), device_id_type=pl.DeviceIdType.MESH)

        def accumulate(ch, s):
            v, _, v2, _, d = geom[ch]
            if s < 3:
                rq = m4(v - d * (s + 1))
                out_ref[qrows(rq), cols[ch]] = (
                    out_ref[qrows(rq), cols[ch]] + comm1[ch][s])
            elif s < 6:
                rp = m4(v2 - d * (s - 2))
                out_ref[prow(qown[ch], rp), cols[ch]] = (
                    out_ref[prow(qown[ch], rp), cols[ch]] + comm2[ch][s - 3])

        def compute_quarter(idx):
            out_ref[qrows(idx), :] = jnp.dot(
                a_ref[qrows(idx), :], b_ref[:, :],
                preferred_element_type=jnp.float32)

        barrier_sem = pltpu.get_barrier_semaphore()
        for nbr in (pr, plq, zr, zl):
            pl.semaphore_signal(barrier_sem, inc=1, device_id=(nbr,),
                                device_id_type=pl.DeviceIdType.MESH)
        pl.semaphore_wait(barrier_sem, 4)

        all_rdmas = []

        def start(ch, s):
            r = step_rdma(ch, s)
            all_rdmas.append(r)
            r.start()
            return r

        compute_quarter(q4)
        live = {}
        live["AR"] = start("AR", 0)
        live["AL"] = start("AL", 0)
        pl.when(z4 != q4)(lambda: compute_quarter(z4))
        live["BR"] = start("BR", 0)
        live["BL"] = start("BL", 0)
        for j in range(4):
            pl.when((j != q4) & (j != z4))(lambda j=j: compute_quarter(j))

        for s in range(NST):
            for ch in CHS:
                live[ch].wait_recv()
                accumulate(ch, s)
                if s + 1 < NST:
                    live[ch] = start(ch, s + 1)

        for r in all_rdmas:
            r.wait_send()

    return pl.pallas_call(
        body,
        out_shape=jax.ShapeDtypeStruct((M, N), jnp.float32),
        in_specs=[
            pl.BlockSpec(memory_space=pltpu.VMEM),
            pl.BlockSpec(memory_space=pltpu.VMEM),
        ],
        out_specs=pl.BlockSpec(memory_space=pltpu.VMEM),
        scratch_shapes=(
            [pltpu.VMEM((3, QROWS, STRIP), jnp.float32) for _ in range(4)]
            + [pltpu.VMEM((3, PROWS, STRIP), jnp.float32) for _ in range(4)]
            + [pltpu.SemaphoreType.DMA((NST,)) for _ in range(8)]
        ),
        compiler_params=pltpu.CompilerParams(collective_id=0),
    )(A, B)
